# baseline (speedup 1.0000x reference)
import functools

import jax
import jax.numpy as jnp
from jax import lax
from jax.experimental import pallas as pl
from jax.experimental.pallas import tpu as pltpu
from jax.experimental.pallas import tpu_sc as plsc

B = 16384
D = 128
L = 16
NC = 2
NS = 16
NW = NC * NS
BPW = B // NW
CHUNK = 64
NCHUNK = BPW // CHUNK
GROUPS = CHUNK // L
NROUND = NCHUNK // 2


def _body(users_hbm, items_hbm, eu_hbm, ei_hbm, out_hbm,
          uidx_v, iidx_v, u0, u1, v0, v1,
          fa, fb, fc, fd, fe, acc_out,
          su0, su1, sv0, sv1):
    wid = lax.axis_index("s") * NC + lax.axis_index("c")
    base = wid * BPW

    ciu = pltpu.async_copy(users_hbm.at[pl.ds(base, BPW)], uidx_v, su0)
    civ = pltpu.async_copy(items_hbm.at[pl.ds(base, BPW)], iidx_v, sv0)
    ciu.wait()
    civ.wait()

    ub = (u0, u1)
    vb = (v0, v1)
    us = (su0, su1)
    vs = (sv0, sv1)

    def start(c, b):
        pltpu.async_copy(
            eu_hbm.at[uidx_v.at[pl.ds(c * CHUNK, CHUNK)]], ub[b], us[b])
        pltpu.async_copy(
            ei_hbm.at[iidx_v.at[pl.ds(c * CHUNK, CHUNK)]], vb[b], vs[b])

    def compute_chunk(cdyn, u_buf, v_buf):
        def group(g, _):
            r0 = g * L
            accs = []
            for r4 in range(0, L, 8):
                for r in range(r4, r4 + 8):
                    pa = (u_buf[r0 + r, pl.ds(0, L)]
                          * v_buf[r0 + r, pl.ds(0, L)])
                    pb = (u_buf[r0 + r, pl.ds(L, L)]
                          * v_buf[r0 + r, pl.ds(L, L)])
                    for j in range(2, D // L, 2):
                        pa = pa + (u_buf[r0 + r, pl.ds(j * L, L)]
                                   * v_buf[r0 + r, pl.ds(j * L, L)])
                        pb = pb + (u_buf[r0 + r, pl.ds((j + 1) * L, L)]
                                   * v_buf[r0 + r, pl.ds((j + 1) * L, L)])
                    accs.append(pa + pb)
                for r in range(r4, r4 + 8):
                    fa[r, pl.ds(0, L)] = accs[r]
            for r in range(L):
                accs[r] = accs[r] + fa[r, pl.ds(8, L)]
                fb[r, pl.ds(0, L)] = accs[r]
            for r in range(L):
                accs[r] = accs[r] + fb[r, pl.ds(4, L)]
                fc[r, pl.ds(0, L)] = accs[r]
            for r in range(L):
                accs[r] = accs[r] + fc[r, pl.ds(2, L)]
                fd[r, pl.ds(0, L)] = accs[r]
            for r in range(L):
                accs[r] = accs[r] + fd[r, pl.ds(1, L)]
            for r in range(L):
                fe[pl.ds(r, L)] = accs[r]
            acc_out[pl.ds(cdyn * CHUNK + r0, L)] = fe[pl.ds(0, L)]
            return 0

        lax.fori_loop(0, GROUPS, group, 0)

    start(0, 0)
    start(1, 1)

    def round_body(s, _):
        for b in range(2):
            cdyn = 2 * s + b
            pltpu.make_async_copy(
                eu_hbm.at[uidx_v.at[pl.ds(cdyn * CHUNK, CHUNK)]],
                ub[b], us[b]).wait()
            pltpu.make_async_copy(
                ei_hbm.at[iidx_v.at[pl.ds(cdyn * CHUNK, CHUNK)]],
                vb[b], vs[b]).wait()

            compute_chunk(cdyn, ub[b], vb[b])

            @pl.when(s + 1 < NROUND)
            def _():
                start(cdyn + 2, b)
        return 0

    lax.fori_loop(0, NROUND, round_body, 0)

    pltpu.sync_copy(acc_out, out_hbm.at[pl.ds(base, BPW)])


def kernel(users, items, embedding_user, embedding_item):
    mesh = plsc.VectorSubcoreMesh(core_axis_name="c", subcore_axis_name="s")
    f = functools.partial(
        pl.kernel,
        out_type=jax.ShapeDtypeStruct((B,), jnp.float32),
        mesh=mesh,
        scratch_types=[
            pltpu.VMEM((BPW,), jnp.int32),
            pltpu.VMEM((BPW,), jnp.int32),
            pltpu.VMEM((CHUNK, D), jnp.float32),
            pltpu.VMEM((CHUNK, D), jnp.float32),
            pltpu.VMEM((CHUNK, D), jnp.float32),
            pltpu.VMEM((CHUNK, D), jnp.float32),
            pltpu.VMEM((L, 24), jnp.float32),
            pltpu.VMEM((L, 24), jnp.float32),
            pltpu.VMEM((L, 24), jnp.float32),
            pltpu.VMEM((L, 24), jnp.float32),
            pltpu.VMEM((2 * L,), jnp.float32),
            pltpu.VMEM((BPW,), jnp.float32),
            pltpu.SemaphoreType.DMA,
            pltpu.SemaphoreType.DMA,
            pltpu.SemaphoreType.DMA,
            pltpu.SemaphoreType.DMA,
        ],
    )(_body)
    return f(users.astype(jnp.int32), items.astype(jnp.int32),
             embedding_user, embedding_item)


# speedup vs baseline: 1.0139x; 1.0139x over previous
import functools

import jax
import jax.numpy as jnp
from jax import lax
from jax.experimental import pallas as pl
from jax.experimental.pallas import tpu as pltpu
from jax.experimental.pallas import tpu_sc as plsc

B = 16384
D = 128
L = 16
NC = 2
NS = 16
NW = NC * NS
BPW = B // NW
CHUNK = 64
NCHUNK = BPW // CHUNK
GROUPS = CHUNK // L
NROUND = NCHUNK // 2


def _body(users_hbm, items_hbm, eu_hbm, ei_hbm, out_hbm,
          uidx_v, iidx_v, u0, u1, v0, v1,
          fa, fb, fc, fd, fe, acc_out,
          su0, su1, sv0, sv1):
    wid = lax.axis_index("s") * NC + lax.axis_index("c")
    base = wid * BPW

    ciu = pltpu.async_copy(users_hbm.at[pl.ds(base, BPW)], uidx_v, su0)
    civ = pltpu.async_copy(items_hbm.at[pl.ds(base, BPW)], iidx_v, sv0)
    ciu.wait()
    civ.wait()

    ub = (u0, u1)
    vb = (v0, v1)
    us = (su0, su1)
    vs = (sv0, sv1)

    H = CHUNK // 2

    def start(c, b):
        pltpu.async_copy(
            eu_hbm.at[uidx_v.at[pl.ds(c * CHUNK, H)]],
            ub[b].at[pl.ds(0, H)], us[b])
        pltpu.async_copy(
            eu_hbm.at[uidx_v.at[pl.ds(c * CHUNK + H, H)]],
            ub[b].at[pl.ds(H, H)], us[b])
        pltpu.async_copy(
            ei_hbm.at[iidx_v.at[pl.ds(c * CHUNK, H)]],
            vb[b].at[pl.ds(0, H)], vs[b])
        pltpu.async_copy(
            ei_hbm.at[iidx_v.at[pl.ds(c * CHUNK + H, H)]],
            vb[b].at[pl.ds(H, H)], vs[b])

    def compute_chunk(cdyn, u_buf, v_buf):
        def group(g, _):
            r0 = g * L
            accs = []
            for r4 in range(0, L, 4):
                for r in range(r4, r4 + 4):
                    pa = (u_buf[r0 + r, pl.ds(0, L)]
                          * v_buf[r0 + r, pl.ds(0, L)])
                    pb = (u_buf[r0 + r, pl.ds(L, L)]
                          * v_buf[r0 + r, pl.ds(L, L)])
                    for j in range(2, D // L, 2):
                        pa = pa + (u_buf[r0 + r, pl.ds(j * L, L)]
                                   * v_buf[r0 + r, pl.ds(j * L, L)])
                        pb = pb + (u_buf[r0 + r, pl.ds((j + 1) * L, L)]
                                   * v_buf[r0 + r, pl.ds((j + 1) * L, L)])
                    accs.append(pa + pb)
                for r in range(r4, r4 + 4):
                    fa[r, pl.ds(0, L)] = accs[r]
            for r in range(L):
                accs[r] = accs[r] + fa[r, pl.ds(8, L)]
                fb[r, pl.ds(0, L)] = accs[r]
            for r in range(L):
                accs[r] = accs[r] + fb[r, pl.ds(4, L)]
                fc[r, pl.ds(0, L)] = accs[r]
            for r in range(L):
                accs[r] = accs[r] + fc[r, pl.ds(2, L)]
                fd[r, pl.ds(0, L)] = accs[r]
            for r in range(L):
                accs[r] = accs[r] + fd[r, pl.ds(1, L)]
            for r in range(L):
                fe[pl.ds(r, L)] = accs[r]
            acc_out[pl.ds(cdyn * CHUNK + r0, L)] = fe[pl.ds(0, L)]
            return 0

        lax.fori_loop(0, GROUPS, group, 0)

    start(0, 0)
    start(1, 1)

    def round_body(s, _):
        for b in range(2):
            cdyn = 2 * s + b
            for h in range(2):
                pltpu.make_async_copy(
                    eu_hbm.at[uidx_v.at[pl.ds(cdyn * CHUNK + h * H, H)]],
                    ub[b].at[pl.ds(h * H, H)], us[b]).wait()
                pltpu.make_async_copy(
                    ei_hbm.at[iidx_v.at[pl.ds(cdyn * CHUNK + h * H, H)]],
                    vb[b].at[pl.ds(h * H, H)], vs[b]).wait()

            compute_chunk(cdyn, ub[b], vb[b])

            @pl.when(s + 1 < NROUND)
            def _():
                start(cdyn + 2, b)
        return 0

    lax.fori_loop(0, NROUND, round_body, 0)

    pltpu.sync_copy(acc_out, out_hbm.at[pl.ds(base, BPW)])


def kernel(users, items, embedding_user, embedding_item):
    mesh = plsc.VectorSubcoreMesh(core_axis_name="c", subcore_axis_name="s")
    f = functools.partial(
        pl.kernel,
        out_type=jax.ShapeDtypeStruct((B,), jnp.float32),
        mesh=mesh,
        scratch_types=[
            pltpu.VMEM((BPW,), jnp.int32),
            pltpu.VMEM((BPW,), jnp.int32),
            pltpu.VMEM((CHUNK, D), jnp.float32),
            pltpu.VMEM((CHUNK, D), jnp.float32),
            pltpu.VMEM((CHUNK, D), jnp.float32),
            pltpu.VMEM((CHUNK, D), jnp.float32),
            pltpu.VMEM((L, 24), jnp.float32),
            pltpu.VMEM((L, 24), jnp.float32),
            pltpu.VMEM((L, 24), jnp.float32),
            pltpu.VMEM((L, 24), jnp.float32),
            pltpu.VMEM((2 * L,), jnp.float32),
            pltpu.VMEM((BPW,), jnp.float32),
            pltpu.SemaphoreType.DMA,
            pltpu.SemaphoreType.DMA,
            pltpu.SemaphoreType.DMA,
            pltpu.SemaphoreType.DMA,
        ],
    )(_body)
    return f(users.astype(jnp.int32), items.astype(jnp.int32),
             embedding_user, embedding_item)


# speedup vs baseline: 1.0200x; 1.0060x over previous
import functools

import jax
import jax.numpy as jnp
from jax import lax
from jax.experimental import pallas as pl
from jax.experimental.pallas import tpu as pltpu
from jax.experimental.pallas import tpu_sc as plsc

B = 16384
D = 128
L = 16
NC = 2
NS = 16
NW = NC * NS
BPW = B // NW
CHUNK = 64
NCHUNK = BPW // CHUNK
GROUPS = CHUNK // L
NROUND = NCHUNK // 2


def _body(users_hbm, items_hbm, eu_hbm, ei_hbm, out_hbm,
          uidx_v, iidx_v, u0, u1, v0, v1,
          fa, fb, fc, fd, fe, acc_out,
          su0, su1, sv0, sv1):
    wid = lax.axis_index("s") * NC + lax.axis_index("c")
    base = wid * BPW

    ciu = pltpu.async_copy(users_hbm.at[pl.ds(base, BPW)], uidx_v, su0)
    civ = pltpu.async_copy(items_hbm.at[pl.ds(base, BPW)], iidx_v, sv0)
    ciu.wait()
    civ.wait()

    ub = (u0, u1)
    vb = (v0, v1)
    us = (su0, su1)
    vs = (sv0, sv1)

    def start(c, b):
        pltpu.async_copy(
            eu_hbm.at[uidx_v.at[pl.ds(c * CHUNK, CHUNK)]], ub[b], us[b])
        pltpu.async_copy(
            ei_hbm.at[iidx_v.at[pl.ds(c * CHUNK, CHUNK)]], vb[b], vs[b])

    def compute_chunk(cdyn, u_buf, v_buf):
        def group(g, _):
            r0 = g * L
            accs = []
            for r4 in range(0, L, 4):
                for r in range(r4, r4 + 4):
                    pa = (u_buf[r0 + r, pl.ds(0, L)]
                          * v_buf[r0 + r, pl.ds(0, L)])
                    pb = (u_buf[r0 + r, pl.ds(L, L)]
                          * v_buf[r0 + r, pl.ds(L, L)])
                    for j in range(2, D // L, 2):
                        pa = pa + (u_buf[r0 + r, pl.ds(j * L, L)]
                                   * v_buf[r0 + r, pl.ds(j * L, L)])
                        pb = pb + (u_buf[r0 + r, pl.ds((j + 1) * L, L)]
                                   * v_buf[r0 + r, pl.ds((j + 1) * L, L)])
                    accs.append(pa + pb)
                for r in range(r4, r4 + 4):
                    fa[r, pl.ds(0, L)] = accs[r]
            for r in range(L):
                accs[r] = accs[r] + fa[r, pl.ds(8, L)]
                fb[r, pl.ds(0, L)] = accs[r]
            for r in range(L):
                accs[r] = accs[r] + fb[r, pl.ds(4, L)]
                fc[r, pl.ds(0, L)] = accs[r]
            for r in range(L):
                accs[r] = accs[r] + fc[r, pl.ds(2, L)]
                fd[r, pl.ds(0, L)] = accs[r]
            for r in range(L):
                accs[r] = accs[r] + fd[r, pl.ds(1, L)]
            for r in range(L):
                fe[pl.ds(r, L)] = accs[r]
            acc_out[pl.ds(cdyn * CHUNK + r0, L)] = fe[pl.ds(0, L)]
            return 0

        lax.fori_loop(0, GROUPS, group, 0)

    start(0, 0)
    start(1, 1)

    def round_body(s, _):
        for b in range(2):
            cdyn = 2 * s + b
            pltpu.make_async_copy(
                eu_hbm.at[uidx_v.at[pl.ds(cdyn * CHUNK, CHUNK)]],
                ub[b], us[b]).wait()
            pltpu.make_async_copy(
                ei_hbm.at[iidx_v.at[pl.ds(cdyn * CHUNK, CHUNK)]],
                vb[b], vs[b]).wait()

            compute_chunk(cdyn, ub[b], vb[b])

            @pl.when(s + 1 < NROUND)
            def _():
                start(cdyn + 2, b)
        return 0

    lax.fori_loop(0, NROUND, round_body, 0)

    pltpu.sync_copy(acc_out, out_hbm.at[pl.ds(base, BPW)])


def kernel(users, items, embedding_user, embedding_item):
    mesh = plsc.VectorSubcoreMesh(core_axis_name="c", subcore_axis_name="s")
    f = functools.partial(
        pl.kernel,
        out_type=jax.ShapeDtypeStruct((B,), jnp.float32),
        mesh=mesh,
        scratch_types=[
            pltpu.VMEM((BPW,), jnp.int32),
            pltpu.VMEM((BPW,), jnp.int32),
            pltpu.VMEM((CHUNK, D), jnp.float32),
            pltpu.VMEM((CHUNK, D), jnp.float32),
            pltpu.VMEM((CHUNK, D), jnp.float32),
            pltpu.VMEM((CHUNK, D), jnp.float32),
            pltpu.VMEM((L, 24), jnp.float32),
            pltpu.VMEM((L, 24), jnp.float32),
            pltpu.VMEM((L, 24), jnp.float32),
            pltpu.VMEM((L, 24), jnp.float32),
            pltpu.VMEM((2 * L,), jnp.float32),
            pltpu.VMEM((BPW,), jnp.float32),
            pltpu.SemaphoreType.DMA,
            pltpu.SemaphoreType.DMA,
            pltpu.SemaphoreType.DMA,
            pltpu.SemaphoreType.DMA,
        ],
    )(_body)
    return f(users.astype(jnp.int32), items.astype(jnp.int32),
             embedding_user, embedding_item)
